# Initial kernel scaffold; baseline (speedup 1.0000x reference)
import jax
import jax.numpy as jnp
from jax.experimental import pallas as pl


def kernel(x, edge_index, batch, r_target, W1_1, b1_1, g_1, be_1, W2_1, b2_1, W1_2, b1_2, g_2, be_2, W2_2, b2_2, W1_3, b1_3, g_3, be_3, W2_3, b2_3, Wh, bh):
    raise NotImplementedError("write your pallas kernel here")


# speedup vs baseline: 2.8871x; 2.8871x over previous
import functools

import jax
import jax.numpy as jnp
from jax import lax
from jax.experimental import pallas as pl
from jax.experimental.pallas import tpu as pltpu
from jax.experimental.pallas import tpu_sc as plsc

_N = 10000
_E = 160000
_H = 256
_G = 64
_T = 4

_NTILES = 16
_NCORES = 2
_K = 64
_NB = 160
_C = 128
_EPT = _K * _NB
_EPAD = _EPT * _NTILES
_NACC = 10240

_R = 1000
_NBLK = _N // _R


def _sc_aggregate(hs, srcs, dsts, zeros, out, src_v, dst_v, rows_v, acc, sem):
    c = lax.axis_index("c")
    s = lax.axis_index("s")
    pltpu.sync_copy(zeros, rows_v)
    for j in range(_NACC // _C // _NTILES):
        pltpu.sync_copy(rows_v, acc.at[pl.ds((s + _NTILES * j) * _C, _C)])
    plsc.subcore_barrier()
    table = hs.at[c]
    half = _NB // 2

    def body(j, carry):
        d0 = pltpu.async_copy(table.at[src_v.at[2 * j]], rows_v.at[pl.ds(0, _K)], sem)
        d1 = pltpu.async_copy(table.at[src_v.at[2 * j + 1]], rows_v.at[pl.ds(_K, _K)], sem)
        d0.wait()
        pltpu.sync_copy(rows_v.at[pl.ds(0, _K)], acc.at[dst_v.at[2 * j]], add=True)
        d1.wait()
        pltpu.sync_copy(rows_v.at[pl.ds(_K, _K)], acc.at[dst_v.at[2 * j + 1]], add=True)
        return carry

    for h in range(2):
        pltpu.sync_copy(srcs.at[s].at[pl.ds(h * half, half)], src_v)
        pltpu.sync_copy(dsts.at[s].at[pl.ds(h * half, half)], dst_v)
        lax.fori_loop(0, half // 2, body, 0)
    plsc.subcore_barrier()
    nfull = _N // _C
    for j in range(5):
        idx = s + _NTILES * j

        @pl.when(idx < nfull)
        def _():
            base = pl.multiple_of(idx * _C, _C)
            pltpu.sync_copy(acc.at[pl.ds(base, _C)], rows_v)
            pltpu.sync_copy(rows_v, out.at[c].at[pl.ds(base, _C)])

    tail = _N - nfull * _C

    @pl.when(s == _NTILES - 1)
    def _():
        pltpu.sync_copy(acc.at[pl.ds(nfull * _C, tail)],
                        rows_v.at[pl.ds(0, tail)])
        pltpu.sync_copy(rows_v.at[pl.ds(0, tail)],
                        out.at[c].at[pl.ds(nfull * _C, tail)])


def _make_sc_agg():
    mesh = plsc.VectorSubcoreMesh(core_axis_name="c", subcore_axis_name="s")
    return pl.kernel(
        _sc_aggregate,
        out_type=jax.ShapeDtypeStruct((_NCORES, _N, 128), jnp.float32),
        mesh=mesh,
        scratch_types=[
            pltpu.VMEM((_NB // 2, _K), jnp.int32),
            pltpu.VMEM((_NB // 2, _K), jnp.int32),
            pltpu.VMEM((2 * _K, 128), jnp.float32),
            pltpu.VMEM_SHARED((_NACC, 128), jnp.float32),
            pltpu.SemaphoreType.DMA,
        ],
    )


def _mlp1_body(xs_ref, agg_ref, w1_ref, b1_ref, y_ref, s1_ref, s2_ref):
    i = pl.program_id(0)
    xa = jnp.concatenate(
        [xs_ref[0] + agg_ref[0], xs_ref[1] + agg_ref[1]], axis=1)
    y = jnp.dot(xa, w1_ref[...], preferred_element_type=jnp.float32) + b1_ref[...]
    y_ref[...] = y

    @pl.when(i == 0)
    def _():
        s1_ref[...] = jnp.zeros_like(s1_ref)
        s2_ref[...] = jnp.zeros_like(s2_ref)

    s1_ref[...] += jnp.sum(y, axis=0, keepdims=True)
    s2_ref[...] += jnp.sum(y * y, axis=0, keepdims=True)


def _mlp1(xs, agg, w1, b1r):
    return pl.pallas_call(
        _mlp1_body,
        grid=(_NBLK,),
        in_specs=[
            pl.BlockSpec((_NCORES, _R, 128), lambda i: (0, i, 0)),
            pl.BlockSpec((_NCORES, _R, 128), lambda i: (0, i, 0)),
            pl.BlockSpec((_H, _H), lambda i: (0, 0)),
            pl.BlockSpec((1, _H), lambda i: (0, 0)),
        ],
        out_specs=[
            pl.BlockSpec((_R, _H), lambda i: (i, 0)),
            pl.BlockSpec((1, _H), lambda i: (0, 0)),
            pl.BlockSpec((1, _H), lambda i: (0, 0)),
        ],
        out_shape=[
            jax.ShapeDtypeStruct((_N, _H), jnp.float32),
            jax.ShapeDtypeStruct((1, _H), jnp.float32),
            jax.ShapeDtypeStruct((1, _H), jnp.float32),
        ],
    )(xs, agg, w1, b1r)


def _mlp2_body(y_ref, s1_ref, s2_ref, g_ref, be_ref, w2_ref, b2_ref, hs_ref):
    ninv = 1.0 / _N
    mean = s1_ref[...] * ninv
    var = s2_ref[...] * ninv - mean * mean
    scale = g_ref[...] * lax.rsqrt(var + 1e-5)
    shift = be_ref[...] - mean * scale
    h = jnp.maximum(y_ref[...] * scale + shift, 0.0)
    o = jnp.dot(h, w2_ref[...], preferred_element_type=jnp.float32) + b2_ref[...]
    o = jnp.maximum(o, 0.0)
    hs_ref[0] = o[:, :128]
    hs_ref[1] = o[:, 128:]


def _mlp2(y, s1, s2, gr, ber, w2, b2r):
    return pl.pallas_call(
        _mlp2_body,
        grid=(_NBLK,),
        in_specs=[
            pl.BlockSpec((_R, _H), lambda i: (i, 0)),
            pl.BlockSpec((1, _H), lambda i: (0, 0)),
            pl.BlockSpec((1, _H), lambda i: (0, 0)),
            pl.BlockSpec((1, _H), lambda i: (0, 0)),
            pl.BlockSpec((1, _H), lambda i: (0, 0)),
            pl.BlockSpec((_H, _H), lambda i: (0, 0)),
            pl.BlockSpec((1, _H), lambda i: (0, 0)),
        ],
        out_specs=pl.BlockSpec((_NCORES, _R, 128), lambda i: (0, i, 0)),
        out_shape=jax.ShapeDtypeStruct((_NCORES, _N, 128), jnp.float32),
    )(y, s1, s2, gr, ber, w2, b2r)


def _pool_body(hs_ref, batch_ref, rt_ref, whm_ref, bh_ref, out_ref, acc_ref):
    i = pl.program_id(0)

    @pl.when(i == 0)
    def _():
        acc_ref[...] = jnp.zeros_like(acc_ref)

    h = jnp.concatenate([hs_ref[0], hs_ref[1]], axis=1)
    onehot = (batch_ref[...] ==
              lax.broadcasted_iota(jnp.int32, (_R, _G), 1)).astype(jnp.float32)
    acc_ref[...] += lax.dot_general(
        onehot, h, (((0,), (0,)), ((), ())),
        preferred_element_type=jnp.float32)

    @pl.when(i == _NBLK - 1)
    def _():
        pooled = acc_ref[...]
        proj = jnp.dot(pooled, whm_ref[...], preferred_element_type=jnp.float32)
        sel = (rt_ref[...] ==
               lax.broadcasted_iota(jnp.int32, (_G, _T), 1)).astype(jnp.float32)
        res = jnp.sum((proj + bh_ref[...]) * sel, axis=1)
        out_ref[...] = res[None, :]


def _pool_head(hs, batch2, rt2, whm, bhr):
    return pl.pallas_call(
        _pool_body,
        grid=(_NBLK,),
        in_specs=[
            pl.BlockSpec((_NCORES, _R, 128), lambda i: (0, i, 0)),
            pl.BlockSpec((_R, 1), lambda i: (i, 0)),
            pl.BlockSpec((_G, 1), lambda i: (0, 0)),
            pl.BlockSpec((_H, _T), lambda i: (0, 0)),
            pl.BlockSpec((1, _T), lambda i: (0, 0)),
        ],
        out_specs=pl.BlockSpec((1, _G), lambda i: (0, 0)),
        out_shape=jax.ShapeDtypeStruct((1, _G), jnp.float32),
        scratch_shapes=[pltpu.VMEM((_G, _H), jnp.float32)],
    )(hs, batch2, rt2, whm, bhr)


def kernel(x, edge_index, batch, r_target, W1_1, b1_1, g_1, be_1, W2_1, b2_1,
           W1_2, b1_2, g_2, be_2, W2_2, b2_2, W1_3, b1_3, g_3, be_3, W2_3,
           b2_3, Wh, bh):
    src = edge_index[0]
    dst = edge_index[1]
    pad = _EPAD - _E
    srcs = jnp.concatenate([src, jnp.zeros((pad,), jnp.int32)]).reshape(
        _NTILES, _NB, _K)
    dsts = jnp.concatenate([dst, jnp.full((pad,), _N, jnp.int32)]).reshape(
        _NTILES, _NB, _K)
    zeros = jnp.zeros((_C, 128), jnp.float32)

    hs = jnp.stack([x[:, :128], x[:, 128:]])
    sc_agg = _make_sc_agg()

    layers = [
        (W1_1, b1_1, g_1, be_1, W2_1, b2_1),
        (W1_2, b1_2, g_2, be_2, W2_2, b2_2),
        (W1_3, b1_3, g_3, be_3, W2_3, b2_3),
    ]
    for (w1, b1, g, be, w2, b2) in layers:
        agg = sc_agg(hs, srcs, dsts, zeros)
        y, s1, s2 = _mlp1(hs, agg, w1, b1[None])
        hs = _mlp2(y, s1, s2, g[None], be[None], w2, b2[None])

    whm = Wh[:, :, 0].T
    bhr = bh[:, 0][None]
    out = _pool_head(hs, batch[:, None], r_target[:, None], whm, bhr)
    return out.reshape(_G)
